# baseline (speedup 1.0000x reference)
import functools

import jax
import jax.numpy as jnp
from jax import lax
from jax.experimental import pallas as pl
from jax.experimental.pallas import tpu as pltpu
from jax.experimental.pallas import tpu_sc as plsc

N = 10000
E = 320000
D = 128
DH = D // 2

NC = 2
NS = 16
CHUNK = 64
NBUF = 8
G_AHEAD = 6
K = -(-E // (NS * CHUNK))
E_PAD = NS * K * CHUNK
ACC_ROWS = N + 8
HIST = N + 16
ROWS_A = 632
ROWS_LAST = N - 15 * ROWS_A
ZROWS_LAST = ACC_ROWS - 15 * ROWS_A


def _zero_chunks(total):
    out, off = [], 0
    while off < total:
        sz = min(CHUNK, total - off)
        out.append((off, sz))
        off += sz
    return out


_SC_PARAMS = pltpu.CompilerParams(needs_layout_passes=False,
                                  use_tc_tiling_on_sc=False)


@functools.cache
def _make_sc_scatter(with_deg: bool):
    mesh = plsc.VectorSubcoreMesh(core_axis_name="c", subcore_axis_name="s",
                                  num_cores=NC, num_subcores=NS)
    out_type = [jax.ShapeDtypeStruct((NC, N, DH), jnp.float32)]
    if with_deg:
        out_type.append(jax.ShapeDtypeStruct((NS * N,), jnp.float32))

    scratch = [
        pltpu.VMEM((K, CHUNK), jnp.int32),
        pltpu.VMEM((K, CHUNK), jnp.int32),
        pltpu.VMEM((NBUF, CHUNK, DH), jnp.float32),
    ]
    if with_deg:
        scratch.append(pltpu.VMEM((HIST,), jnp.float32))
    scratch += [
        pltpu.VMEM_SHARED((ACC_ROWS, DH), jnp.float32),
        pltpu.SemaphoreType.DMA,
        pltpu.SemaphoreType.DMA,
    ]

    def body(x_h, src_hbm, dst_hbm, *rest):
        if with_deg:
            (s_out, deg_out, src_v, dst_v, rows_v, hist_v,
             acc_sh, gsem, ssem) = rest
        else:
            (s_out, src_v, dst_v, rows_v,
             acc_sh, gsem, ssem) = rest
            deg_out = hist_v = None

        cid = lax.axis_index("c")
        sid = lax.axis_index("s")

        zeros16 = jnp.zeros((16,), jnp.float32)
        ones16 = jnp.ones((16,), jnp.float32)

        @pl.loop(0, CHUNK)
        def _(i):
            for j in range(DH // 16):
                rows_v[0, i, pl.ds(j * 16, 16)] = zeros16

        if with_deg:
            @pl.loop(0, HIST // 16)
            def _(i):
                hist_v[pl.ds(i * 16, 16)] = zeros16

        pltpu.sync_copy(src_hbm.at[sid], src_v)
        pltpu.sync_copy(dst_hbm.at[sid], dst_v)

        @pl.when(sid < NS - 1)
        def _():
            base = sid * ROWS_A
            for off, sz in _zero_chunks(ROWS_A):
                pltpu.sync_copy(rows_v.at[0, pl.ds(0, sz)],
                                acc_sh.at[pl.ds(base + off, sz)])

        @pl.when(sid == NS - 1)
        def _():
            base = (NS - 1) * ROWS_A
            for off, sz in _zero_chunks(ZROWS_LAST):
                pltpu.sync_copy(rows_v.at[0, pl.ds(0, sz)],
                                acc_sh.at[pl.ds(base + off, sz)])

        plsc.subcore_barrier()

        def edge_loop(xref, hist):
            def g_start(j, b):
                pltpu.async_copy(xref.at[src_v.at[j]], rows_v.at[b], gsem)

            def s_start(j, b):
                pltpu.async_copy(rows_v.at[b], acc_sh.at[dst_v.at[j]],
                                 ssem, add=True)

            def wait_chunk(sem):
                pltpu.make_async_copy(xref.at[pl.ds(0, CHUNK)],
                                      rows_v.at[0], sem).wait()

            W = NBUF - G_AHEAD
            for p in range(G_AHEAD):
                g_start(p, p)

            @pl.loop(0, K)
            def _(j):
                wait_chunk(gsem)
                s_start(j, lax.rem(j, NBUF))
                if hist is not None:
                    for g in range(CHUNK // 16):
                        idx = dst_v[j, pl.ds(g * 16, 16)]
                        plsc.addupdate_scatter(hist, [idx], ones16)

                jn = j + G_AHEAD
                @pl.when(jn < K)
                def _():
                    @pl.when(j >= W)
                    def _():
                        wait_chunk(ssem)
                    g_start(jn, lax.rem(jn, NBUF))

            for _i in range(NBUF):
                wait_chunk(ssem)

        @pl.when(cid == 0)
        def _():
            edge_loop(x_h.at[0], hist_v)

        @pl.when(cid == 1)
        def _():
            edge_loop(x_h.at[1], None)

        plsc.subcore_barrier()

        @pl.when(sid < NS - 1)
        def _():
            base = sid * ROWS_A
            pltpu.sync_copy(acc_sh.at[pl.ds(base, ROWS_A)],
                            s_out.at[cid, pl.ds(base, ROWS_A)])

        @pl.when(sid == NS - 1)
        def _():
            base = (NS - 1) * ROWS_A
            pltpu.sync_copy(acc_sh.at[pl.ds(base, ROWS_LAST)],
                            s_out.at[cid, pl.ds(base, ROWS_LAST)])

        if with_deg:
            @pl.when(cid == 0)
            def _():
                pltpu.sync_copy(hist_v.at[pl.ds(0, N)],
                                deg_out.at[pl.ds(sid * N, N)])

    return pl.kernel(body, out_type=tuple(out_type), mesh=mesh,
                     scratch_types=scratch, compiler_params=_SC_PARAMS,
                     name=f"sage_scatter{'_deg' if with_deg else ''}")


def _tc_dense_body(full_in, full_out, x_ref, s_ref, deg_ref, ws_ref, wn_ref,
                   b_ref, o_ref):
    deg = jnp.sum(deg_ref[...], axis=1, keepdims=True)
    rinv = 1.0 / jnp.maximum(deg, 1.0)
    dot = functools.partial(jnp.dot, preferred_element_type=jnp.float32)
    if full_in:
        self_part = dot(x_ref[...], ws_ref[...])
    else:
        self_part = (dot(x_ref[0], ws_ref[0:DH, :])
                     + dot(x_ref[1], ws_ref[DH:D, :]))
    neigh = (dot(s_ref[0] * rinv, wn_ref[0:DH, :])
             + dot(s_ref[1] * rinv, wn_ref[DH:D, :]))
    o = self_part + neigh + b_ref[...]
    if full_out:
        o_ref[...] = o
    else:
        o_ref[0] = o[:, 0:DH]
        o_ref[1] = o[:, DH:D]


_TC_R = 1000


def _tc_dense(x, s_h, deg_t, w_self, w_neigh, b, full_in, full_out):
    x_spec = (pl.BlockSpec((_TC_R, D), lambda i: (i, 0)) if full_in
              else pl.BlockSpec((NC, _TC_R, DH), lambda i: (0, i, 0)))
    if full_out:
        out_spec = pl.BlockSpec((_TC_R, D), lambda i: (i, 0))
        out_shape = jax.ShapeDtypeStruct((N, D), jnp.float32)
    else:
        out_spec = pl.BlockSpec((NC, _TC_R, DH), lambda i: (0, i, 0))
        out_shape = jax.ShapeDtypeStruct((NC, N, DH), jnp.float32)
    return pl.pallas_call(
        functools.partial(_tc_dense_body, full_in, full_out),
        grid=(N // _TC_R,),
        in_specs=[
            x_spec,
            pl.BlockSpec((NC, _TC_R, DH), lambda i: (0, i, 0)),
            pl.BlockSpec((_TC_R, NS), lambda i: (i, 0)),
            pl.BlockSpec((D, D), lambda i: (0, 0)),
            pl.BlockSpec((D, D), lambda i: (0, 0)),
            pl.BlockSpec((1, D), lambda i: (0, 0)),
        ],
        out_specs=out_spec,
        out_shape=out_shape,
    )(x, s_h, deg_t, w_self, w_neigh, b)


@jax.jit
def kernel(h, edge_index, W_self1, W_neigh1, b1, W_self2, W_neigh2, b2):
    src = edge_index[0].astype(jnp.int32)
    dst = edge_index[1].astype(jnp.int32)
    pad = E_PAD - E
    src_t = jnp.concatenate([src, jnp.zeros((pad,), jnp.int32)]
                            ).reshape(NS, K, CHUNK)
    dst_t = jnp.concatenate([dst, jnp.full((pad,), N, jnp.int32)]
                            ).reshape(NS, K, CHUNK)

    h_h = jnp.stack([h[:, 0:DH], h[:, DH:D]])

    s1_h, deg_flat = _make_sc_scatter(True)(h_h, src_t, dst_t)
    deg_t = deg_flat.reshape(NS, N).T
    out1_h = _tc_dense(h, s1_h, deg_t, W_self1, W_neigh1, b1.reshape(1, D),
                       full_in=True, full_out=False)
    (s2_h,) = _make_sc_scatter(False)(out1_h, src_t, dst_t)
    out2 = _tc_dense(out1_h, s2_h, deg_t, W_self2, W_neigh2,
                     b2.reshape(1, D), full_in=False, full_out=True)
    return out2


# speedup vs baseline: 1.0072x; 1.0072x over previous
import functools

import jax
import jax.numpy as jnp
from jax import lax
from jax.experimental import pallas as pl
from jax.experimental.pallas import tpu as pltpu
from jax.experimental.pallas import tpu_sc as plsc

N = 10000
E = 320000
D = 128
DH = D // 2

NC = 2
NS = 16
CHUNK = 64
NBUF = 8
G_AHEAD = 6
K = -(-E // (NS * CHUNK))
E_PAD = NS * K * CHUNK
ACC_ROWS = N + 8
HIST = N + 16
ROWS_A = 632
ROWS_LAST = N - 15 * ROWS_A
ZROWS_LAST = ACC_ROWS - 15 * ROWS_A


def _zero_chunks(total):
    out, off = [], 0
    while off < total:
        sz = min(CHUNK, total - off)
        out.append((off, sz))
        off += sz
    return out


_SC_PARAMS = pltpu.CompilerParams(needs_layout_passes=False,
                                  use_tc_tiling_on_sc=False)


@functools.cache
def _make_sc_scatter(with_deg: bool):
    mesh = plsc.VectorSubcoreMesh(core_axis_name="c", subcore_axis_name="s",
                                  num_cores=NC, num_subcores=NS)
    out_type = [jax.ShapeDtypeStruct((NC, N, DH), jnp.float32)]
    if with_deg:
        out_type.append(jax.ShapeDtypeStruct((NS * N,), jnp.float32))

    scratch = [
        pltpu.VMEM((K, CHUNK), jnp.int32),
        pltpu.VMEM((K, CHUNK), jnp.int32),
        pltpu.VMEM((NBUF, CHUNK, DH), jnp.float32),
    ]
    if with_deg:
        scratch.append(pltpu.VMEM((HIST,), jnp.float32))
    scratch += [
        pltpu.VMEM_SHARED((ACC_ROWS, DH), jnp.float32),
        pltpu.SemaphoreType.DMA,
        pltpu.SemaphoreType.DMA,
    ]

    def body(x_h, src_hbm, dst_hbm, *rest):
        if with_deg:
            (s_out, deg_out, src_v, dst_v, rows_v, hist_v,
             acc_sh, gsem, ssem) = rest
        else:
            (s_out, src_v, dst_v, rows_v,
             acc_sh, gsem, ssem) = rest
            deg_out = hist_v = None

        cid = lax.axis_index("c")
        sid = lax.axis_index("s")

        zeros16 = jnp.zeros((16,), jnp.float32)
        ones16 = jnp.ones((16,), jnp.float32)

        @pl.loop(0, CHUNK)
        def _(i):
            for j in range(DH // 16):
                rows_v[0, i, pl.ds(j * 16, 16)] = zeros16

        if with_deg:
            @pl.loop(0, HIST // 16)
            def _(i):
                hist_v[pl.ds(i * 16, 16)] = zeros16

        pltpu.sync_copy(src_hbm.at[sid], src_v)
        pltpu.sync_copy(dst_hbm.at[sid], dst_v)

        @pl.when(sid < NS - 1)
        def _():
            base = sid * ROWS_A
            for off, sz in _zero_chunks(ROWS_A):
                pltpu.sync_copy(rows_v.at[0, pl.ds(0, sz)],
                                acc_sh.at[pl.ds(base + off, sz)])

        @pl.when(sid == NS - 1)
        def _():
            base = (NS - 1) * ROWS_A
            for off, sz in _zero_chunks(ZROWS_LAST):
                pltpu.sync_copy(rows_v.at[0, pl.ds(0, sz)],
                                acc_sh.at[pl.ds(base + off, sz)])

        plsc.subcore_barrier()

        def edge_loop(xref, hist):
            def g_start(j, b):
                pltpu.async_copy(xref.at[src_v.at[j]], rows_v.at[b], gsem)

            def s_start(j, b):
                pltpu.async_copy(rows_v.at[b], acc_sh.at[dst_v.at[j]],
                                 ssem, add=True)

            def wait_chunk(sem):
                pltpu.make_async_copy(xref.at[pl.ds(0, CHUNK)],
                                      rows_v.at[0], sem).wait()

            W = NBUF - G_AHEAD
            for p in range(G_AHEAD):
                g_start(p, p)

            @pl.loop(0, K)
            def _(j):
                wait_chunk(gsem)
                s_start(j, lax.rem(j, NBUF))
                if hist is not None:
                    for g in range(CHUNK // 16):
                        idx = dst_v[j, pl.ds(g * 16, 16)]
                        plsc.addupdate_scatter(hist, [idx], ones16)

                jn = j + G_AHEAD
                @pl.when(jn < K)
                def _():
                    @pl.when(j >= W)
                    def _():
                        wait_chunk(ssem)
                    g_start(jn, lax.rem(jn, NBUF))

            for _i in range(NBUF):
                wait_chunk(ssem)

        @pl.when(cid == 0)
        def _():
            edge_loop(x_h.at[0], hist_v)

        @pl.when(cid == 1)
        def _():
            edge_loop(x_h.at[1], None)

        plsc.subcore_barrier()

        @pl.when(sid < NS - 1)
        def _():
            base = sid * ROWS_A
            pltpu.sync_copy(acc_sh.at[pl.ds(base, ROWS_A)],
                            s_out.at[cid, pl.ds(base, ROWS_A)])

        @pl.when(sid == NS - 1)
        def _():
            base = (NS - 1) * ROWS_A
            pltpu.sync_copy(acc_sh.at[pl.ds(base, ROWS_LAST)],
                            s_out.at[cid, pl.ds(base, ROWS_LAST)])

        if with_deg:
            @pl.when(cid == 0)
            def _():
                pltpu.sync_copy(hist_v.at[pl.ds(0, N)],
                                deg_out.at[pl.ds(sid * N, N)])

    return pl.kernel(body, out_type=tuple(out_type), mesh=mesh,
                     scratch_types=scratch, compiler_params=_SC_PARAMS,
                     name=f"sage_scatter{'_deg' if with_deg else ''}")


_TC_R = 1000
_DOT = functools.partial(jnp.dot, preferred_element_type=jnp.float32)


def _tc_self_body(full_in, x_ref, ws_ref, b_ref, o_ref):
    if full_in:
        o_ref[...] = _DOT(x_ref[...], ws_ref[...]) + b_ref[...]
    else:
        o_ref[...] = (_DOT(x_ref[0], ws_ref[0:DH, :])
                      + _DOT(x_ref[1], ws_ref[DH:D, :]) + b_ref[...])


def _tc_self(x, w_self, b, full_in):
    x_spec = (pl.BlockSpec((_TC_R, D), lambda i: (i, 0)) if full_in
              else pl.BlockSpec((NC, _TC_R, DH), lambda i: (0, i, 0)))
    return pl.pallas_call(
        functools.partial(_tc_self_body, full_in),
        grid=(N // _TC_R,),
        in_specs=[
            x_spec,
            pl.BlockSpec((D, D), lambda i: (0, 0)),
            pl.BlockSpec((1, D), lambda i: (0, 0)),
        ],
        out_specs=pl.BlockSpec((_TC_R, D), lambda i: (i, 0)),
        out_shape=jax.ShapeDtypeStruct((N, D), jnp.float32),
    )(x, w_self, b)


def _tc_neigh_body(full_out, self_ref, s_ref, deg_ref, wn_ref, o_ref):
    deg = jnp.sum(deg_ref[...], axis=1, keepdims=True)
    rinv = 1.0 / jnp.maximum(deg, 1.0)
    o = (self_ref[...]
         + _DOT(s_ref[0] * rinv, wn_ref[0:DH, :])
         + _DOT(s_ref[1] * rinv, wn_ref[DH:D, :]))
    if full_out:
        o_ref[...] = o
    else:
        o_ref[0] = o[:, 0:DH]
        o_ref[1] = o[:, DH:D]


def _tc_neigh(self_full, s_h, deg_t, w_neigh, full_out):
    if full_out:
        out_spec = pl.BlockSpec((_TC_R, D), lambda i: (i, 0))
        out_shape = jax.ShapeDtypeStruct((N, D), jnp.float32)
    else:
        out_spec = pl.BlockSpec((NC, _TC_R, DH), lambda i: (0, i, 0))
        out_shape = jax.ShapeDtypeStruct((NC, N, DH), jnp.float32)
    return pl.pallas_call(
        functools.partial(_tc_neigh_body, full_out),
        grid=(N // _TC_R,),
        in_specs=[
            pl.BlockSpec((_TC_R, D), lambda i: (i, 0)),
            pl.BlockSpec((NC, _TC_R, DH), lambda i: (0, i, 0)),
            pl.BlockSpec((_TC_R, NS), lambda i: (i, 0)),
            pl.BlockSpec((D, D), lambda i: (0, 0)),
        ],
        out_specs=out_spec,
        out_shape=out_shape,
    )(self_full, s_h, deg_t, w_neigh)


@jax.jit
def kernel(h, edge_index, W_self1, W_neigh1, b1, W_self2, W_neigh2, b2):
    src = edge_index[0].astype(jnp.int32)
    dst = edge_index[1].astype(jnp.int32)
    pad = E_PAD - E
    src_t = jnp.concatenate([src, jnp.zeros((pad,), jnp.int32)]
                            ).reshape(NS, K, CHUNK)
    dst_t = jnp.concatenate([dst, jnp.full((pad,), N, jnp.int32)]
                            ).reshape(NS, K, CHUNK)

    h_h = jnp.stack([h[:, 0:DH], h[:, DH:D]])

    s1_h, deg_flat = _make_sc_scatter(True)(h_h, src_t, dst_t)
    self1 = _tc_self(h, W_self1, b1.reshape(1, D), full_in=True)
    deg_t = deg_flat.reshape(NS, N).T
    out1_h = _tc_neigh(self1, s1_h, deg_t, W_neigh1, full_out=False)
    (s2_h,) = _make_sc_scatter(False)(out1_h, src_t, dst_t)
    self2 = _tc_self(out1_h, W_self2, b2.reshape(1, D), full_in=False)
    out2 = _tc_neigh(self2, s2_h, deg_t, W_neigh2, full_out=True)
    return out2


# speedup vs baseline: 1.0239x; 1.0166x over previous
import functools

import jax
import jax.numpy as jnp
from jax import lax
from jax.experimental import pallas as pl
from jax.experimental.pallas import tpu as pltpu
from jax.experimental.pallas import tpu_sc as plsc

N = 10000
E = 320000
D = 128
DH = D // 2

NC = 2
NS = 16
CHUNK = 64
NBUF = 8
G_AHEAD = 6
K = -(-E // (NS * CHUNK))
E_PAD = NS * K * CHUNK
ACC_ROWS = N + 8
HIST = N + 16
ROWS_A = 632
ROWS_LAST = N - 15 * ROWS_A
ZROWS_LAST = ACC_ROWS - 15 * ROWS_A


def _zero_chunks(total):
    out, off = [], 0
    while off < total:
        sz = min(CHUNK, total - off)
        out.append((off, sz))
        off += sz
    return out


_SC_PARAMS = pltpu.CompilerParams(needs_layout_passes=False,
                                  use_tc_tiling_on_sc=False)


@functools.cache
def _make_sc_scatter(with_deg: bool):
    mesh = plsc.VectorSubcoreMesh(core_axis_name="c", subcore_axis_name="s",
                                  num_cores=NC, num_subcores=NS)
    out_type = [jax.ShapeDtypeStruct((NC, N, DH), jnp.float32)]
    if with_deg:
        out_type.append(jax.ShapeDtypeStruct((NS * N,), jnp.float32))

    scratch = [
        pltpu.VMEM((K, CHUNK), jnp.int32),
        pltpu.VMEM((K, CHUNK), jnp.int32),
        pltpu.VMEM((NBUF, CHUNK, DH), jnp.float32),
    ]
    if with_deg:
        scratch.append(pltpu.VMEM((HIST,), jnp.float32))
    scratch += [
        pltpu.VMEM_SHARED((ACC_ROWS, DH), jnp.float32),
        pltpu.SemaphoreType.DMA,
        pltpu.SemaphoreType.DMA,
    ]

    def body(x_h, src_hbm, dst_hbm, *rest):
        if with_deg:
            (s_out, deg_out, src_v, dst_v, rows_v, hist_v,
             acc_sh, gsem, ssem) = rest
        else:
            (s_out, src_v, dst_v, rows_v,
             acc_sh, gsem, ssem) = rest
            deg_out = hist_v = None

        cid = lax.axis_index("c")
        sid = lax.axis_index("s")

        zeros16 = jnp.zeros((16,), jnp.float32)
        ones16 = jnp.ones((16,), jnp.float32)

        @pl.loop(0, CHUNK)
        def _(i):
            for j in range(DH // 16):
                rows_v[0, i, pl.ds(j * 16, 16)] = zeros16

        if with_deg:
            @pl.loop(0, HIST // 16)
            def _(i):
                hist_v[pl.ds(i * 16, 16)] = zeros16

        pltpu.sync_copy(src_hbm.at[sid], src_v)
        pltpu.sync_copy(dst_hbm.at[sid], dst_v)

        @pl.when(sid < NS - 1)
        def _():
            base = sid * ROWS_A
            for off, sz in _zero_chunks(ROWS_A):
                pltpu.sync_copy(rows_v.at[0, pl.ds(0, sz)],
                                acc_sh.at[pl.ds(base + off, sz)])

        @pl.when(sid == NS - 1)
        def _():
            base = (NS - 1) * ROWS_A
            for off, sz in _zero_chunks(ZROWS_LAST):
                pltpu.sync_copy(rows_v.at[0, pl.ds(0, sz)],
                                acc_sh.at[pl.ds(base + off, sz)])

        plsc.subcore_barrier()

        def edge_loop(xref, hist):
            def g_start(j, b):
                pltpu.async_copy(xref.at[src_v.at[j]], rows_v.at[b], gsem)

            def s_start(j, b):
                pltpu.async_copy(rows_v.at[b], acc_sh.at[dst_v.at[j]],
                                 ssem, add=True)

            def wait_chunk(sem):
                pltpu.make_async_copy(xref.at[pl.ds(0, CHUNK)],
                                      rows_v.at[0], sem).wait()

            W = NBUF - G_AHEAD
            for p in range(G_AHEAD):
                g_start(p, p)

            @pl.loop(0, K)
            def _(j):
                wait_chunk(gsem)
                s_start(j, lax.rem(j, NBUF))
                if hist is not None:
                    for g in range(CHUNK // 16):
                        idx = dst_v[j, pl.ds(g * 16, 16)]
                        plsc.addupdate_scatter(hist, [idx], ones16)

                jn = j + G_AHEAD
                @pl.when(jn < K)
                def _():
                    @pl.when(j >= W)
                    def _():
                        wait_chunk(ssem)
                    g_start(jn, lax.rem(jn, NBUF))

            for _i in range(NBUF):
                wait_chunk(ssem)

        @pl.when(cid == 0)
        def _():
            edge_loop(x_h.at[0], hist_v)

        @pl.when(cid == 1)
        def _():
            edge_loop(x_h.at[1], None)

        plsc.subcore_barrier()

        @pl.when(sid < NS - 1)
        def _():
            base = sid * ROWS_A
            pltpu.sync_copy(acc_sh.at[pl.ds(base, ROWS_A)],
                            s_out.at[cid, pl.ds(base, ROWS_A)])

        @pl.when(sid == NS - 1)
        def _():
            base = (NS - 1) * ROWS_A
            pltpu.sync_copy(acc_sh.at[pl.ds(base, ROWS_LAST)],
                            s_out.at[cid, pl.ds(base, ROWS_LAST)])

        if with_deg:
            @pl.when(cid == 0)
            def _():
                pltpu.sync_copy(hist_v.at[pl.ds(0, N)],
                                deg_out.at[pl.ds(sid * N, N)])

    return pl.kernel(body, out_type=tuple(out_type), mesh=mesh,
                     scratch_types=scratch, compiler_params=_SC_PARAMS,
                     name=f"sage_scatter{'_deg' if with_deg else ''}")


_TC_R = 2000
_DOT = functools.partial(jnp.dot, preferred_element_type=jnp.float32)


def _tc_self_body(full_in, x_ref, ws_ref, b_ref, o_ref):
    if full_in:
        o_ref[...] = _DOT(x_ref[...], ws_ref[...]) + b_ref[...]
    else:
        o_ref[...] = (_DOT(x_ref[0], ws_ref[0:DH, :])
                      + _DOT(x_ref[1], ws_ref[DH:D, :]) + b_ref[...])


def _tc_self(x, w_self, b, full_in):
    x_spec = (pl.BlockSpec((_TC_R, D), lambda i: (i, 0)) if full_in
              else pl.BlockSpec((NC, _TC_R, DH), lambda i: (0, i, 0)))
    return pl.pallas_call(
        functools.partial(_tc_self_body, full_in),
        grid=(N // _TC_R,),
        in_specs=[
            x_spec,
            pl.BlockSpec((D, D), lambda i: (0, 0)),
            pl.BlockSpec((1, D), lambda i: (0, 0)),
        ],
        out_specs=pl.BlockSpec((_TC_R, D), lambda i: (i, 0)),
        out_shape=jax.ShapeDtypeStruct((N, D), jnp.float32),
    )(x, w_self, b)


def _tc_neigh_body(full_out, self_ref, s_ref, deg_ref, wn_ref, o_ref):
    deg = jnp.sum(deg_ref[...], axis=1, keepdims=True)
    rinv = 1.0 / jnp.maximum(deg, 1.0)
    o = (self_ref[...]
         + _DOT(s_ref[0] * rinv, wn_ref[0:DH, :])
         + _DOT(s_ref[1] * rinv, wn_ref[DH:D, :]))
    if full_out:
        o_ref[...] = o
    else:
        o_ref[0] = o[:, 0:DH]
        o_ref[1] = o[:, DH:D]


def _tc_neigh(self_full, s_h, deg_t, w_neigh, full_out):
    if full_out:
        out_spec = pl.BlockSpec((_TC_R, D), lambda i: (i, 0))
        out_shape = jax.ShapeDtypeStruct((N, D), jnp.float32)
    else:
        out_spec = pl.BlockSpec((NC, _TC_R, DH), lambda i: (0, i, 0))
        out_shape = jax.ShapeDtypeStruct((NC, N, DH), jnp.float32)
    return pl.pallas_call(
        functools.partial(_tc_neigh_body, full_out),
        grid=(N // _TC_R,),
        in_specs=[
            pl.BlockSpec((_TC_R, D), lambda i: (i, 0)),
            pl.BlockSpec((NC, _TC_R, DH), lambda i: (0, i, 0)),
            pl.BlockSpec((_TC_R, NS), lambda i: (i, 0)),
            pl.BlockSpec((D, D), lambda i: (0, 0)),
        ],
        out_specs=out_spec,
        out_shape=out_shape,
    )(self_full, s_h, deg_t, w_neigh)


@jax.jit
def kernel(h, edge_index, W_self1, W_neigh1, b1, W_self2, W_neigh2, b2):
    src = edge_index[0].astype(jnp.int32)
    dst = edge_index[1].astype(jnp.int32)
    pad = E_PAD - E
    src_t = jnp.concatenate([src, jnp.zeros((pad,), jnp.int32)]
                            ).reshape(NS, K, CHUNK)
    dst_t = jnp.concatenate([dst, jnp.full((pad,), N, jnp.int32)]
                            ).reshape(NS, K, CHUNK)

    h_h = jnp.stack([h[:, 0:DH], h[:, DH:D]])

    s1_h, deg_flat = _make_sc_scatter(True)(h_h, src_t, dst_t)
    self1 = _tc_self(h, W_self1, b1.reshape(1, D), full_in=True)
    deg_t = deg_flat.reshape(NS, N).T
    out1_h = _tc_neigh(self1, s1_h, deg_t, W_neigh1, full_out=False)
    (s2_h,) = _make_sc_scatter(False)(out1_h, src_t, dst_t)
    self2 = _tc_self(out1_h, W_self2, b2.reshape(1, D), full_in=False)
    out2 = _tc_neigh(self2, s2_h, deg_t, W_neigh2, full_out=True)
    return out2
